# baseline (speedup 1.0000x reference)
import functools

import jax
import jax.numpy as jnp
import numpy as np
from jax import lax
from jax.experimental import pallas as pl
from jax.experimental.pallas import tpu as pltpu
from jax.experimental.pallas import tpu_sc as plsc

VOCAB = 1000000
D = 128
B = 16384
NC = 2
NS = 16
NW = NC * NS
B_PER_W = B // NW
CHUNK = 128
N_CHUNKS = B_PER_W // CHUNK


def _sc_gather_sum(emb_table, idx0, idx1):
  mesh = plsc.VectorSubcoreMesh(core_axis_name="c", subcore_axis_name="s")

  @functools.partial(
      pl.kernel,
      out_type=jax.ShapeDtypeStruct((B, D // 2), jnp.int32),
      mesh=mesh,
      scratch_types=[
          pltpu.VMEM((CHUNK,), jnp.int32),
          pltpu.VMEM((CHUNK,), jnp.int32),
          pltpu.VMEM((CHUNK,), jnp.int32),
          pltpu.VMEM((CHUNK,), jnp.int32),
          pltpu.VMEM((CHUNK, D), jnp.float32),
          pltpu.VMEM((CHUNK, D), jnp.float32),
          pltpu.VMEM((CHUNK, D), jnp.float32),
          pltpu.VMEM((CHUNK, D), jnp.float32),
          pltpu.VMEM((CHUNK, D // 2), jnp.int32),
          pltpu.VMEM((CHUNK, D // 2), jnp.int32),
          pltpu.SemaphoreType.DMA,
          pltpu.SemaphoreType.DMA,
          pltpu.SemaphoreType.DMA,
          pltpu.SemaphoreType.DMA,
          pltpu.SemaphoreType.DMA,
          pltpu.SemaphoreType.DMA,
      ],
  )
  def k(table_hbm, idx0_hbm, idx1_hbm, out_hbm,
        i0a, i0b, i1a, i1b, r0a, r0b, r1a, r1b, sm_a, sm_b,
        ga0, ga1, gb0, gb1, wa, wb_sem):
    idx0_v = (i0a, i0b)
    idx1_v = (i1a, i1b)
    rows0 = (r0a, r0b)
    rows1 = (r1a, r1b)
    sums = (sm_a, sm_b)
    gsem0 = (ga0, gb0)
    gsem1 = (ga1, gb1)
    wsems = (wa, wb_sem)
    wid = lax.axis_index("s") * NC + lax.axis_index("c")
    base_w = wid * B_PER_W

    def start_gathers(g):
      slot = g % 2
      base = base_w + g * CHUNK
      pltpu.sync_copy(idx0_hbm.at[pl.ds(base, CHUNK)], idx0_v[slot])
      pltpu.sync_copy(idx1_hbm.at[pl.ds(base, CHUNK)], idx1_v[slot])
      c0 = pltpu.async_copy(table_hbm.at[idx0_v[slot]], rows0[slot],
                            gsem0[slot])
      c1 = pltpu.async_copy(table_hbm.at[idx1_v[slot]], rows1[slot],
                            gsem1[slot])
      return (c0, c1)

    cps = [start_gathers(0)]
    wbs = [None, None]
    for g in range(N_CHUNKS):
      slot = g % 2
      if g + 1 < N_CHUNKS:
        cps.append(start_gathers(g + 1))
      cps[g][0].wait()
      cps[g][1].wait()
      r0 = rows0[slot]
      r1 = rows1[slot]
      sbuf = sums[slot]
      if wbs[slot] is not None:
        wbs[slot].wait()

      def add_body(s, c2, r0=r0, r1=r1, sbuf=sbuf):
        for c in range(D // 32):
          sl_a = pl.ds(c * 32, 16)
          sl_b = pl.ds(c * 32 + 16, 16)
          va = r0[s, sl_a] + r1[s, sl_a]
          vb = r0[s, sl_b] + r1[s, sl_b]
          ai = lax.bitcast_convert_type(va, jnp.int32)
          bi = lax.bitcast_convert_type(vb, jnp.int32)
          ar = ai + (jnp.int32(0x7FFF) + ((ai >> 16) & jnp.int32(1)))
          br = bi + (jnp.int32(0x7FFF) + ((bi >> 16) & jnp.int32(1)))
          lo = (ar >> 16) & jnp.int32(0xFFFF)
          hi = br & jnp.int32(-65536)
          sbuf[s, pl.ds(c * 16, 16)] = lo | hi
        return c2

      lax.fori_loop(0, CHUNK, add_body, 0, unroll=False)
      wbs[slot] = pltpu.async_copy(
          sbuf, out_hbm.at[pl.ds(base_w + g * CHUNK, CHUNK)], wsems[slot])
    for wb in wbs:
      if wb is not None:
        wb.wait()

  return k(emb_table, idx0, idx1)


BLK = 4096


def _mlp_body(s_ref, w1t_ref, b1_ref, w2t_ref, b2_ref, o_ref):
  s = s_ref[...].astype(jnp.float32)
  w1t = w1t_ref[...]
  ht = lax.dot_general(w1t, s, (((1,), (1,)), ((), ())),
                       preferred_element_type=jnp.float32)
  ht = jnp.maximum(ht + b1_ref[...], 0.0)
  zt = lax.dot_general(w2t_ref[...], ht, (((1,), (0,)), ((), ())),
                       preferred_element_type=jnp.float32)
  zt = zt + b2_ref[0, 0]
  o_ref[...] = jnp.maximum(jax.nn.sigmoid(zt), 0.0)


def _tc_mlp(s, W1t, b1, W2t, b2):
  grid = (B // BLK,)
  return pl.pallas_call(
      _mlp_body,
      grid=grid,
      in_specs=[
          pl.BlockSpec((BLK, D), lambda i: (i, 0)),
          pl.BlockSpec((16, D), lambda i: (0, 0)),
          pl.BlockSpec((16, 1), lambda i: (0, 0)),
          pl.BlockSpec((1, 16), lambda i: (0, 0)),
          pl.BlockSpec((1, 1), lambda i: (0, 0)),
      ],
      out_specs=pl.BlockSpec((1, BLK), lambda i: (0, i)),
      out_shape=jax.ShapeDtypeStruct((1, B), jnp.float32),
  )(s, W1t, b1, W2t, b2)


_PERM = np.empty(D, np.int32)
for _c in range(D // 32):
  for _i in range(16):
    _PERM[32 * _c + 2 * _i] = 32 * _c + _i
    _PERM[32 * _c + 2 * _i + 1] = 32 * _c + 16 + _i


def kernel(input_tensor, emb_table, W1, b1, W2, b2):
  idxT = input_tensor.astype(jnp.int32).T
  s_i32 = _sc_gather_sum(emb_table, idxT[0], idxT[1])
  s = jax.lax.bitcast_convert_type(s_i32, jnp.bfloat16).reshape(B, D)
  out = _tc_mlp(s, 0.5 * W1[_PERM, :].T, b1.reshape(16, 1),
                W2.reshape(1, 16), b2.reshape(1, 1))
  return out.reshape(B, 1)


# speedup vs baseline: 2.0693x; 2.0693x over previous
import functools

import jax
import jax.numpy as jnp
import numpy as np
from jax import lax
from jax.experimental import pallas as pl
from jax.experimental.pallas import tpu as pltpu
from jax.experimental.pallas import tpu_sc as plsc

VOCAB = 1000000
D = 128
B = 16384
NC = 2
NS = 16
NW = NC * NS
B_PER_W = B // NW
CHUNK = 128
N_CHUNKS = B_PER_W // CHUNK


def _sc_gather_sum(emb_table, idx0, idx1):
  mesh = plsc.VectorSubcoreMesh(core_axis_name="c", subcore_axis_name="s")

  @functools.partial(
      pl.kernel,
      out_type=jax.ShapeDtypeStruct((B, D // 2), jnp.int32),
      mesh=mesh,
      scratch_types=[
          pltpu.VMEM((CHUNK,), jnp.int32),
          pltpu.VMEM((CHUNK,), jnp.int32),
          pltpu.VMEM((CHUNK,), jnp.int32),
          pltpu.VMEM((CHUNK,), jnp.int32),
          pltpu.VMEM((CHUNK, D), jnp.float32),
          pltpu.VMEM((CHUNK, D), jnp.float32),
          pltpu.VMEM((CHUNK, D), jnp.float32),
          pltpu.VMEM((CHUNK, D), jnp.float32),
          pltpu.VMEM((CHUNK, D // 2), jnp.int32),
          pltpu.VMEM((CHUNK, D // 2), jnp.int32),
          pltpu.SemaphoreType.DMA,
          pltpu.SemaphoreType.DMA,
          pltpu.SemaphoreType.DMA,
          pltpu.SemaphoreType.DMA,
          pltpu.SemaphoreType.DMA,
          pltpu.SemaphoreType.DMA,
      ],
  )
  def k(table_hbm, idx0_hbm, idx1_hbm, out_hbm,
        i0a, i0b, i1a, i1b, r0a, r0b, r1a, r1b, sm_a, sm_b,
        ga0, ga1, gb0, gb1, wa, wb_sem):
    idx0_v = (i0a, i0b)
    idx1_v = (i1a, i1b)
    rows0 = (r0a, r0b)
    rows1 = (r1a, r1b)
    sums = (sm_a, sm_b)
    gsem0 = (ga0, gb0)
    gsem1 = (ga1, gb1)
    wsems = (wa, wb_sem)
    wid = lax.axis_index("s") * NC + lax.axis_index("c")
    base_w = wid * B_PER_W

    def start_gathers(g):
      slot = g % 2
      base = base_w + g * CHUNK
      pltpu.sync_copy(idx0_hbm.at[pl.ds(base, CHUNK)], idx0_v[slot])
      pltpu.sync_copy(idx1_hbm.at[pl.ds(base, CHUNK)], idx1_v[slot])
      c0 = pltpu.async_copy(table_hbm.at[idx0_v[slot]], rows0[slot],
                            gsem0[slot])
      c1 = pltpu.async_copy(table_hbm.at[idx1_v[slot]], rows1[slot],
                            gsem1[slot])
      return (c0, c1)

    cps = [start_gathers(0)]
    wbs = [None, None]
    for g in range(N_CHUNKS):
      slot = g % 2
      if g + 1 < N_CHUNKS:
        cps.append(start_gathers(g + 1))
      cps[g][0].wait()
      cps[g][1].wait()
      r0 = rows0[slot]
      r1 = rows1[slot]
      sbuf = sums[slot]
      if wbs[slot] is not None:
        wbs[slot].wait()

      def add_body(s, c2, r0=r0, r1=r1, sbuf=sbuf):
        for c in range(D // 32):
          sl_a = pl.ds(c * 32, 16)
          sl_b = pl.ds(c * 32 + 16, 16)
          va = r0[s, sl_a] + r1[s, sl_a]
          vb = r0[s, sl_b] + r1[s, sl_b]
          ai = lax.bitcast_convert_type(va, jnp.int32)
          bi = lax.bitcast_convert_type(vb, jnp.int32)
          ar = ai + (jnp.int32(0x7FFF) + ((ai >> 16) & jnp.int32(1)))
          br = bi + (jnp.int32(0x7FFF) + ((bi >> 16) & jnp.int32(1)))
          lo = (ar >> 16) & jnp.int32(0xFFFF)
          hi = br & jnp.int32(-65536)
          sbuf[s, pl.ds(c * 16, 16)] = lo | hi
        return c2

      lax.fori_loop(0, CHUNK, add_body, 0, unroll=False)
      wbs[slot] = pltpu.async_copy(
          sbuf, out_hbm.at[pl.ds(base_w + g * CHUNK, CHUNK)], wsems[slot])
    for wb in wbs:
      if wb is not None:
        wb.wait()

  return k(emb_table, idx0, idx1)


BLK = 4096


def _mlp_body(si_ref, w1tl_ref, w1th_ref, b1_ref, w2t_ref, b2_ref, o_ref):
  si = si_ref[...]
  f_lo = lax.bitcast_convert_type(si << 16, jnp.float32)
  f_hi = lax.bitcast_convert_type(si & jnp.int32(-65536), jnp.float32)
  ht = lax.dot_general(w1tl_ref[...], f_lo, (((1,), (1,)), ((), ())),
                       preferred_element_type=jnp.float32)
  ht = ht + lax.dot_general(w1th_ref[...], f_hi, (((1,), (1,)), ((), ())),
                            preferred_element_type=jnp.float32)
  ht = jnp.maximum(ht + b1_ref[...], 0.0)
  zt = lax.dot_general(w2t_ref[...], ht, (((1,), (0,)), ((), ())),
                       preferred_element_type=jnp.float32)
  zt = zt + b2_ref[0, 0]
  o_ref[...] = jnp.maximum(jax.nn.sigmoid(zt), 0.0)


def _tc_mlp(s, W1tl, W1th, b1, W2t, b2):
  grid = (B // BLK,)
  return pl.pallas_call(
      _mlp_body,
      grid=grid,
      in_specs=[
          pl.BlockSpec((BLK, D // 2), lambda i: (i, 0)),
          pl.BlockSpec((16, D // 2), lambda i: (0, 0)),
          pl.BlockSpec((16, D // 2), lambda i: (0, 0)),
          pl.BlockSpec((16, 1), lambda i: (0, 0)),
          pl.BlockSpec((1, 16), lambda i: (0, 0)),
          pl.BlockSpec((1, 1), lambda i: (0, 0)),
      ],
      out_specs=pl.BlockSpec((1, BLK), lambda i: (0, i)),
      out_shape=jax.ShapeDtypeStruct((1, B), jnp.float32),
  )(s, W1tl, W1th, b1, W2t, b2)


_PERM = np.empty(D, np.int32)
for _c in range(D // 32):
  for _i in range(16):
    _PERM[32 * _c + 2 * _i] = 32 * _c + _i
    _PERM[32 * _c + 2 * _i + 1] = 32 * _c + 16 + _i


def kernel(input_tensor, emb_table, W1, b1, W2, b2):
  idxT = input_tensor.astype(jnp.int32).T
  s_i32 = _sc_gather_sum(emb_table, idxT[0], idxT[1])
  out = _tc_mlp(s_i32, 0.5 * W1[_PERM[0::2], :].T, 0.5 * W1[_PERM[1::2], :].T,
                b1.reshape(16, 1), W2.reshape(1, 16), b2.reshape(1, 1))
  return out.reshape(B, 1)


# speedup vs baseline: 2.1279x; 1.0283x over previous
import functools

import jax
import jax.numpy as jnp
import numpy as np
from jax import lax
from jax.experimental import pallas as pl
from jax.experimental.pallas import tpu as pltpu
from jax.experimental.pallas import tpu_sc as plsc

VOCAB = 1000000
D = 128
B = 16384
NC = 2
NS = 16
NW = NC * NS
B_PER_W = B // NW
CHUNK = 128
N_CHUNKS = B_PER_W // CHUNK


def _sc_gather_sum(emb_table, idx0, idx1):
  mesh = plsc.VectorSubcoreMesh(core_axis_name="c", subcore_axis_name="s")

  @functools.partial(
      pl.kernel,
      out_type=jax.ShapeDtypeStruct((B, D), jnp.float32),
      mesh=mesh,
      scratch_types=[
          pltpu.VMEM((CHUNK,), jnp.int32),
          pltpu.VMEM((CHUNK,), jnp.int32),
          pltpu.VMEM((CHUNK,), jnp.int32),
          pltpu.VMEM((CHUNK,), jnp.int32),
          pltpu.VMEM((CHUNK, D), jnp.float32),
          pltpu.VMEM((CHUNK, D), jnp.float32),
          pltpu.VMEM((CHUNK, D), jnp.float32),
          pltpu.VMEM((CHUNK, D), jnp.float32),
          pltpu.VMEM((CHUNK, D), jnp.float32),
          pltpu.VMEM((CHUNK, D), jnp.float32),
          pltpu.SemaphoreType.DMA,
          pltpu.SemaphoreType.DMA,
          pltpu.SemaphoreType.DMA,
          pltpu.SemaphoreType.DMA,
          pltpu.SemaphoreType.DMA,
          pltpu.SemaphoreType.DMA,
      ],
  )
  def k(table_hbm, idx0_hbm, idx1_hbm, out_hbm,
        i0a, i0b, i1a, i1b, r0a, r0b, r1a, r1b, sm_a, sm_b,
        ga0, ga1, gb0, gb1, wa, wb_sem):
    idx0_v = (i0a, i0b)
    idx1_v = (i1a, i1b)
    rows0 = (r0a, r0b)
    rows1 = (r1a, r1b)
    sums = (sm_a, sm_b)
    gsem0 = (ga0, gb0)
    gsem1 = (ga1, gb1)
    wsems = (wa, wb_sem)
    wid = lax.axis_index("s") * NC + lax.axis_index("c")
    base_w = wid * B_PER_W

    def start_gathers(g):
      slot = g % 2
      base = base_w + g * CHUNK
      pltpu.sync_copy(idx0_hbm.at[pl.ds(base, CHUNK)], idx0_v[slot])
      pltpu.sync_copy(idx1_hbm.at[pl.ds(base, CHUNK)], idx1_v[slot])
      c0 = pltpu.async_copy(table_hbm.at[idx0_v[slot]], rows0[slot],
                            gsem0[slot])
      c1 = pltpu.async_copy(table_hbm.at[idx1_v[slot]], rows1[slot],
                            gsem1[slot])
      return (c0, c1)

    cps = [start_gathers(0)]
    wbs = [None, None]
    for g in range(N_CHUNKS):
      slot = g % 2
      if g + 1 < N_CHUNKS:
        cps.append(start_gathers(g + 1))
      cps[g][0].wait()
      cps[g][1].wait()
      r0 = rows0[slot]
      r1 = rows1[slot]
      sbuf = sums[slot]
      if wbs[slot] is not None:
        wbs[slot].wait()

      def add_body(s, c2, r0=r0, r1=r1, sbuf=sbuf):
        for c in range(D // 16):
          sl = pl.ds(c * 16, 16)
          sbuf[s, sl] = r0[s, sl] + r1[s, sl]
        return c2

      lax.fori_loop(0, CHUNK, add_body, 0, unroll=False)
      wbs[slot] = pltpu.async_copy(
          sbuf, out_hbm.at[pl.ds(base_w + g * CHUNK, CHUNK)], wsems[slot])
    for wb in wbs:
      if wb is not None:
        wb.wait()

  return k(emb_table, idx0, idx1)


BLK = 4096


def _mlp_body(s_ref, w1t_ref, b1_ref, w2t_ref, b2_ref, o_ref):
  s = s_ref[...]
  w1t = w1t_ref[...]
  ht = lax.dot_general(w1t, s, (((1,), (1,)), ((), ())),
                       preferred_element_type=jnp.float32)
  ht = jnp.maximum(ht + b1_ref[...], 0.0)
  zt = lax.dot_general(w2t_ref[...], ht, (((1,), (0,)), ((), ())),
                       preferred_element_type=jnp.float32)
  zt = zt + b2_ref[0, 0]
  o_ref[...] = jnp.maximum(jax.nn.sigmoid(zt), 0.0)


def _tc_mlp(s, W1t, b1, W2t, b2):
  grid = (B // BLK,)
  return pl.pallas_call(
      _mlp_body,
      grid=grid,
      in_specs=[
          pl.BlockSpec((BLK, D), lambda i: (i, 0)),
          pl.BlockSpec((16, D), lambda i: (0, 0)),
          pl.BlockSpec((16, 1), lambda i: (0, 0)),
          pl.BlockSpec((1, 16), lambda i: (0, 0)),
          pl.BlockSpec((1, 1), lambda i: (0, 0)),
      ],
      out_specs=pl.BlockSpec((1, BLK), lambda i: (0, i)),
      out_shape=jax.ShapeDtypeStruct((1, B), jnp.float32),
  )(s, W1t, b1, W2t, b2)


def kernel(input_tensor, emb_table, W1, b1, W2, b2):
  idxT = input_tensor.astype(jnp.int32).T
  s = _sc_gather_sum(emb_table, idxT[0], idxT[1])
  out = _tc_mlp(s, 0.5 * W1.T, b1.reshape(16, 1), W2.reshape(1, 16),
                b2.reshape(1, 1))
  return out.reshape(B, 1)
